# baseline (speedup 1.0000x reference)
import functools

import jax
import jax.numpy as jnp
import numpy as np
from jax import lax
from jax.experimental import pallas as pl
from jax.experimental.pallas import tpu as pltpu
from jax.experimental.pallas import tpu_sc as plsc

NF = 3
DPF = 2
C = 32
H = 1536
W = 512

NC = 2
NS = 16
NW = NC * NS
CHUNK = 128
NG = CHUNK // 16
NTAP = NF * 4

_HF = np.float32((H // NF - 1.0) / (H - 1.0))
_OFF = tuple(
    np.float32(v) for v in
    (np.linspace(np.float32(0.0), np.float32((NF - 1) * (H // NF)), NF)
     .astype(np.float32) / np.float32(H - 1.0)) * np.float32(2.0) - np.float32(1.0)
)


def _transpose_body(in_ref, out_ref):
    out_ref[...] = in_ref[...].T


def _transpose_table(t2d):
    hw = t2d.shape[1]
    tblk = 2048
    return pl.pallas_call(
        _transpose_body,
        grid=(hw // tblk,),
        in_specs=[pl.BlockSpec((C, tblk), lambda j: (0, j))],
        out_specs=pl.BlockSpec((tblk, C), lambda j: (j, 0)),
        out_shape=jax.ShapeDtypeStruct((hw, C), jnp.float32),
    )(t2d)


def _xpart(u):
    gx = u * 2.0 - 1.0
    ix = ((gx + 1.0) * W - 1.0) * 0.5
    x0 = (ix + 1.0).astype(jnp.int32) - 1
    fx = ix - x0.astype(jnp.float32)
    v0 = (x0 >= 0) & (x0 <= W - 1)
    v1 = (x0 + 1 >= 0) & (x0 + 1 <= W - 1)
    wx0 = jnp.where(v0, 1.0 - fx, 0.0)
    wx1 = jnp.where(v1, fx, 0.0)
    x0c = jnp.clip(x0, 0, W - 1)
    x1c = jnp.clip(x0 + 1, 0, W - 1)
    return x0c, x1c, wx0, wx1


def _ypart(v, f):
    gn = v * 2.0 - 1.0
    gy = (gn + 1.0) * _HF + _OFF[f]
    iy = ((gy + 1.0) * H - 1.0) * 0.5
    y0 = (iy + 1.0).astype(jnp.int32) - 1
    fy = iy - y0.astype(jnp.float32)
    v0 = (y0 >= 0) & (y0 <= H - 1)
    v1 = (y0 + 1 >= 0) & (y0 + 1 <= H - 1)
    wy0 = jnp.where(v0, 1.0 - fy, 0.0)
    wy1 = jnp.where(v1, fy, 0.0)
    y0c = jnp.clip(y0, 0, H - 1)
    y1c = jnp.clip(y0 + 1, 0, H - 1)
    return y0c, y1c, wy0, wy1


def _sc_body(xt_hbm, table_hbm, out_hbm, x_v, idx_v, w_v, rows_v, out_v,
             sem0, sem1, semx, semo0, semo1):
    npts = out_hbm.shape[1]
    ppw = npts // NW
    nchunk = ppw // CHUNK
    wid = lax.axis_index("s") * NC + lax.axis_index("c")
    wbase = wid * ppw

    def pass_a(buf):
        @plsc.parallel_loop(0, NG)
        def pa(g):
            sl = pl.ds(g * 16, 16)
            xa = x_v[buf, 0, sl]
            xb = x_v[buf, 1, sl]
            xc = x_v[buf, 2, sl]
            xp01 = _xpart(xa)
            xp2 = _xpart(xb)
            xps = (xp01, xp01, xp2)
            yps = (_ypart(xb, 0), _ypart(xc, 1), _ypart(xc, 2))
            for f in range(NF):
                x0c, x1c, wx0, wx1 = xps[f]
                y0c, y1c, wy0, wy1 = yps[f]
                r0 = y0c * W
                r1 = y1c * W
                idxs = (r0 + x0c, r0 + x1c, r1 + x0c, r1 + x1c)
                wts = (wx0 * wy0, wx1 * wy0, wx0 * wy1, wx1 * wy1)
                for t in range(4):
                    idx_v[buf, 4 * f + t, sl] = idxs[t]
                    w_v[buf, 4 * f + t, sl] = wts[t]

    def fetch_x(start, buf):
        return pltpu.async_copy(
            xt_hbm.at[:, pl.ds(start, CHUNK)], x_v.at[buf], semx)

    def issue(buf, sem):
        return [
            pltpu.async_copy(
                table_hbm.at[idx_v.at[buf, t]], rows_v.at[buf, t], sem)
            for t in range(NTAP)
        ]

    def combine(start, buf, semo):
        @plsc.parallel_loop(0, NG, unroll=2)
        def pc(g):
            base = g * 16
            sl = pl.ds(base, 16)
            lane = lax.iota(jnp.int32, 16)
            ri = base + lane
            ws = [w_v[buf, t, sl] for t in range(NTAP)]
            for ch in range(C):
                cs = jnp.bitwise_and(lane + ch, C - 1)
                prod = None
                for f in range(NF):
                    acc = None
                    for t in range(4):
                        val = plsc.load_gather(
                            rows_v.at[buf, 4 * f + t], [ri, cs])
                        term = ws[4 * f + t] * val
                        acc = term if acc is None else acc + term
                    prod = acc if prod is None else prod * acc
                plsc.store_scatter(out_v.at[buf], [cs, ri], prod)

        return pltpu.async_copy(
            out_v.at[buf], out_hbm.at[:, pl.ds(start, CHUNK)], semo)

    pltpu.sync_copy(xt_hbm.at[:, pl.ds(wbase, CHUNK)], x_v.at[0])
    pass_a(0)

    def body2(j, _):
        s0 = wbase + (2 * j) * CHUNK
        s1 = s0 + CHUNK
        s2w = wbase + jnp.remainder(2 * j + 2, nchunk) * CHUNK
        xd1 = fetch_x(s1, 1)
        d0 = issue(0, sem0)
        xd1.wait()
        pass_a(1)
        d1 = issue(1, sem1)
        xd2 = fetch_x(s2w, 0)
        for d in d0:
            d.wait()
        od0 = combine(s0, 0, semo0)
        xd2.wait()
        pass_a(0)
        for d in d1:
            d.wait()
        od1 = combine(s1, 1, semo1)
        od0.wait()
        od1.wait()
        return 0

    lax.fori_loop(0, nchunk // 2, body2, 0)


def _sc_lookup(xt, table, npts):
    mesh = plsc.VectorSubcoreMesh(
        core_axis_name="c", subcore_axis_name="s", num_cores=NC, num_subcores=NS)
    return pl.kernel(
        _sc_body,
        out_type=jax.ShapeDtypeStruct((C, npts), jnp.float32),
        mesh=mesh,
        compiler_params=pltpu.CompilerParams(
            needs_layout_passes=False, use_tc_tiling_on_sc=False),
        scratch_types=[
            pltpu.VMEM((2, NF, CHUNK), jnp.float32),
            pltpu.VMEM((2, NTAP, CHUNK), jnp.int32),
            pltpu.VMEM((2, NTAP, CHUNK), jnp.float32),
            pltpu.VMEM((2, NTAP, CHUNK, C), jnp.float32),
            pltpu.VMEM((2, C, CHUNK), jnp.float32),
            pltpu.SemaphoreType.DMA,
            pltpu.SemaphoreType.DMA,
            pltpu.SemaphoreType.DMA,
            pltpu.SemaphoreType.DMA,
            pltpu.SemaphoreType.DMA,
        ],
    )(xt, table)


def kernel(x, tensor):
    npts = x.shape[0]
    assert npts % (NW * CHUNK) == 0
    table = tensor.reshape(C, H * W).T
    xt = x.reshape(npts, NF).T
    return _sc_lookup(xt, table, npts).T


# speedup vs baseline: 1.2523x; 1.2523x over previous
import jax
import jax.numpy as jnp
import numpy as np
from jax import lax
from jax.experimental import pallas as pl
from jax.experimental.pallas import tpu as pltpu
from jax.experimental.pallas import tpu_sc as plsc

NF = 3
DPF = 2
C = 32
H = 1536
W = 512

NC = 2
NS = 16
NW = NC * NS
CHUNK = 128
NG = CHUNK // 16
NTAP = NF * 4

_HF = np.float32((H // NF - 1.0) / (H - 1.0))
_OFF = tuple(
    np.float32(v) for v in
    (np.linspace(np.float32(0.0), np.float32((NF - 1) * (H // NF)), NF)
     .astype(np.float32) / np.float32(H - 1.0)) * np.float32(2.0) - np.float32(1.0)
)


def _xpart(u):
    gx = u * 2.0 - 1.0
    ix = ((gx + 1.0) * W - 1.0) * 0.5
    x0 = (ix + 1.0).astype(jnp.int32) - 1
    fx = ix - x0.astype(jnp.float32)
    v0 = (x0 >= 0) & (x0 <= W - 1)
    v1 = (x0 + 1 >= 0) & (x0 + 1 <= W - 1)
    wx0 = jnp.where(v0, 1.0 - fx, 0.0)
    wx1 = jnp.where(v1, fx, 0.0)
    x0c = jnp.clip(x0, 0, W - 1)
    x1c = jnp.clip(x0 + 1, 0, W - 1)
    return x0c, x1c, wx0, wx1


def _ypart(v, f):
    gn = v * 2.0 - 1.0
    gy = (gn + 1.0) * _HF + _OFF[f]
    iy = ((gy + 1.0) * H - 1.0) * 0.5
    y0 = (iy + 1.0).astype(jnp.int32) - 1
    fy = iy - y0.astype(jnp.float32)
    v0 = (y0 >= 0) & (y0 <= H - 1)
    v1 = (y0 + 1 >= 0) & (y0 + 1 <= H - 1)
    wy0 = jnp.where(v0, 1.0 - fy, 0.0)
    wy1 = jnp.where(v1, fy, 0.0)
    y0c = jnp.clip(y0, 0, H - 1)
    y1c = jnp.clip(y0 + 1, 0, H - 1)
    return y0c, y1c, wy0, wy1


def _sc_body(xt_hbm, table_hbm, out_hbm, x_v, idx_v, w_v, rows_v, out_v,
             sem0, sem1, semx, semo0, semo1):
    npts = out_hbm.shape[1]
    ppw = npts // NW
    nchunk = ppw // CHUNK
    wid = lax.axis_index("s") * NC + lax.axis_index("c")
    wbase = wid * ppw

    def pass_a(buf):
        @plsc.parallel_loop(0, NG)
        def pa(g):
            sl = pl.ds(g * 16, 16)
            xa = x_v[buf, 0, sl]
            xb = x_v[buf, 1, sl]
            xc = x_v[buf, 2, sl]
            xp01 = _xpart(xa)
            xp2 = _xpart(xb)
            xps = (xp01, xp01, xp2)
            yps = (_ypart(xb, 0), _ypart(xc, 1), _ypart(xc, 2))
            for f in range(NF):
                x0c, x1c, wx0, wx1 = xps[f]
                y0c, y1c, wy0, wy1 = yps[f]
                r0 = y0c * W
                r1 = y1c * W
                idxs = (r0 + x0c, r0 + x1c, r1 + x0c, r1 + x1c)
                wts = (wx0 * wy0, wx1 * wy0, wx0 * wy1, wx1 * wy1)
                for t in range(4):
                    idx_v[buf, 4 * f + t, sl] = idxs[t]
                    w_v[buf, 4 * f + t, sl] = wts[t]

    def fetch_x(start, buf):
        return pltpu.async_copy(
            xt_hbm.at[:, pl.ds(start, CHUNK)], x_v.at[buf], semx)

    def issue(buf, sem):
        return [
            pltpu.async_copy(
                table_hbm.at[idx_v.at[buf, t]], rows_v.at[buf, t], sem)
            for t in range(NTAP)
        ]

    def combine(start, buf, semo):
        @plsc.parallel_loop(0, NG)
        def pc(g):
            base = g * 16
            sl = pl.ds(base, 16)
            lane = lax.iota(jnp.int32, 16)
            ri = base + lane
            ws = [w_v[buf, t, sl] for t in range(NTAP)]
            for ch in range(C):
                cs = jnp.bitwise_and(lane + ch, C - 1)
                prod = None
                for f in range(NF):
                    acc = None
                    for t in range(4):
                        val = plsc.load_gather(
                            rows_v.at[buf, 4 * f + t], [ri, cs])
                        term = ws[4 * f + t] * val
                        acc = term if acc is None else acc + term
                    prod = acc if prod is None else prod * acc
                plsc.store_scatter(out_v.at[buf], [cs, ri], prod)

        return pltpu.async_copy(
            out_v.at[buf], out_hbm.at[:, pl.ds(start, CHUNK)], semo)

    pltpu.sync_copy(xt_hbm.at[:, pl.ds(wbase, CHUNK)], x_v.at[0])
    pass_a(0)

    def body2(j, _):
        s0 = wbase + (2 * j) * CHUNK
        s1 = s0 + CHUNK
        s2w = wbase + jnp.remainder(2 * j + 2, nchunk) * CHUNK
        xd1 = fetch_x(s1, 1)
        d0 = issue(0, sem0)
        xd1.wait()
        pass_a(1)
        d1 = issue(1, sem1)
        xd2 = fetch_x(s2w, 0)
        for d in d0:
            d.wait()
        od0 = combine(s0, 0, semo0)
        xd2.wait()
        pass_a(0)
        for d in d1:
            d.wait()
        od1 = combine(s1, 1, semo1)
        od0.wait()
        od1.wait()
        return 0

    lax.fori_loop(0, nchunk // 2, body2, 0)


def _sc_lookup(xt, table, npts):
    mesh = plsc.VectorSubcoreMesh(
        core_axis_name="c", subcore_axis_name="s", num_cores=NC, num_subcores=NS)
    return pl.kernel(
        _sc_body,
        out_type=jax.ShapeDtypeStruct((C, npts), jnp.float32),
        mesh=mesh,
        compiler_params=pltpu.CompilerParams(
            needs_layout_passes=False, use_tc_tiling_on_sc=False),
        scratch_types=[
            pltpu.VMEM((2, NF, CHUNK), jnp.float32),
            pltpu.VMEM((2, NTAP, CHUNK), jnp.int32),
            pltpu.VMEM((2, NTAP, CHUNK), jnp.float32),
            pltpu.VMEM((2, NTAP, CHUNK, C), jnp.float32),
            pltpu.VMEM((2, C, CHUNK), jnp.float32),
            pltpu.SemaphoreType.DMA,
            pltpu.SemaphoreType.DMA,
            pltpu.SemaphoreType.DMA,
            pltpu.SemaphoreType.DMA,
            pltpu.SemaphoreType.DMA,
        ],
    )(xt, table)


def kernel(x, tensor):
    npts = x.shape[0]
    assert npts % (NW * CHUNK) == 0
    table = tensor.reshape(C, H * W).T
    xt = x.reshape(npts, NF).T
    return _sc_lookup(xt, table, npts).T
